# Initial kernel scaffold; baseline (speedup 1.0000x reference)
import jax
import jax.numpy as jnp
from jax.experimental import pallas as pl


def kernel(edge_index, particle_class, particle_charge, particle_pos, particle_mom, particle_energy, pflow_class_logits, pflow_charge_logits, pflow_pos, pflow_mom, pflow_energy, predicted_setsizes, particle_batch, pflow_batch):
    raise NotImplementedError("write your pallas kernel here")


# speedup vs baseline: 1.0219x; 1.0219x over previous
import functools

import jax
import jax.numpy as jnp
from jax.experimental import pallas as pl

_NP = 100000
_NF = 100000
_E = 1600000
_B = 1000
_S = 256

_BLKE = 32000
_G = _E // _BLKE


def _pair_loss_body(f_ref, o_ref):
    f = f_ref[...]

    cl = f[0:5, :]
    m = jnp.max(cl, axis=0, keepdims=True)
    lse_c = m + jnp.log(jnp.sum(jnp.exp(cl - m), axis=0, keepdims=True))
    tgt_c = f[5:6, :].astype(jnp.int32)
    iota5 = jax.lax.broadcasted_iota(jnp.int32, (5, _BLKE), 0)
    chosen_c = jnp.sum(jnp.where(iota5 == tgt_c, cl, 0.0), axis=0, keepdims=True)
    class_loss = lse_c - chosen_c

    ch = f[6:9, :]
    m2 = jnp.max(ch, axis=0, keepdims=True)
    lse_h = m2 + jnp.log(jnp.sum(jnp.exp(ch - m2), axis=0, keepdims=True))
    tgt_h = f[9:10, :].astype(jnp.int32)
    iota3 = jax.lax.broadcasted_iota(jnp.int32, (3, _BLKE), 0)
    chosen_h = jnp.sum(jnp.where(iota3 == tgt_h, ch, 0.0), axis=0, keepdims=True)
    charge_loss = lse_h - chosen_h

    dpos = f[10:13, :] - f[13:16, :]
    pos_loss = jnp.sqrt(jnp.sum(dpos * dpos, axis=0, keepdims=True))
    dmom = f[16:19, :] - f[19:22, :]
    mom_loss = jnp.sqrt(jnp.sum(dmom * dmom, axis=0, keepdims=True))

    de = f[23:24, :] - f[22:23, :]
    energy_loss = de * de

    o_ref[...] = (class_loss + charge_loss + pos_loss + mom_loss + energy_loss)[0, :][None, :]


@functools.partial(jax.jit, static_argnums=())
def _pair_loss(feat):
    out = pl.pallas_call(
        _pair_loss_body,
        grid=(_G,),
        in_specs=[pl.BlockSpec((24, _BLKE), lambda i: (0, i))],
        out_specs=pl.BlockSpec((1, _BLKE), lambda i: (0, i)),
        out_shape=jax.ShapeDtypeStruct((1, _E), jnp.float32),
    )(feat)
    return out.reshape(_E)


def kernel(edge_index, particle_class, particle_charge, particle_pos,
           particle_mom, particle_energy, pflow_class_logits,
           pflow_charge_logits, pflow_pos, pflow_mom, pflow_energy,
           predicted_setsizes, particle_batch, pflow_batch):
    src = edge_index[0]
    dst = edge_index[1]

    feat = jnp.concatenate([
        pflow_class_logits[dst].T,
        particle_class[src].astype(jnp.float32)[None],
        pflow_charge_logits[dst].T,
        particle_charge[src].astype(jnp.float32)[None],
        particle_pos[src].T,
        pflow_pos[dst].T,
        particle_mom[src].T,
        pflow_mom[dst].T,
        particle_energy[src][None],
        pflow_energy[dst][None],
    ], axis=0)

    pair_loss = _pair_loss(feat)

    min_pflow = jax.ops.segment_min(pair_loss, dst, num_segments=_NF)
    min_pflow = jnp.where(jnp.isfinite(min_pflow), min_pflow, 0.0)
    min_part = jax.ops.segment_min(pair_loss, src, num_segments=_NP)
    min_part = jnp.where(jnp.isfinite(min_part), min_part, 0.0)

    particle_loss = jax.ops.segment_sum(min_part, particle_batch, num_segments=_B).mean()
    pflow_loss = jax.ops.segment_sum(min_pflow, pflow_batch, num_segments=_B).mean()

    n_per_event = jnp.bincount(particle_batch, length=_B)
    pf_counts = jnp.bincount(pflow_batch, length=_B)
    mean_setsizes = jax.ops.segment_sum(predicted_setsizes, pflow_batch, num_segments=_B)
    mean_setsizes = mean_setsizes / jnp.maximum(pf_counts, 1)[:, None].astype(jnp.float32)
    tgt = jnp.clip(n_per_event, 0, _S - 1)
    logp = jax.nn.log_softmax(mean_setsizes, axis=-1)
    setsize_loss = (-jnp.take_along_axis(logp, tgt[:, None], axis=-1)[:, 0]).mean()

    loss = particle_loss + pflow_loss + setsize_loss
    return loss, particle_loss, setsize_loss, pflow_loss


# speedup vs baseline: 6.4739x; 6.3351x over previous
import functools

import jax
import jax.numpy as jnp
from jax.experimental import pallas as pl

_NP = 100000
_NF = 100000
_E = 1600000
_B = 1000
_S = 256

_BLKE = 32000
_G = _E // _BLKE


def _pair_loss_body(f_ref, o_ref):
    f = f_ref[...]

    cl = f[0:5, :]
    m = jnp.max(cl, axis=0, keepdims=True)
    lse_c = m + jnp.log(jnp.sum(jnp.exp(cl - m), axis=0, keepdims=True))
    tgt_c = f[15:16, :].astype(jnp.int32)
    iota5 = jax.lax.broadcasted_iota(jnp.int32, (5, _BLKE), 0)
    chosen_c = jnp.sum(jnp.where(iota5 == tgt_c, cl, 0.0), axis=0, keepdims=True)
    class_loss = lse_c - chosen_c

    ch = f[5:8, :]
    m2 = jnp.max(ch, axis=0, keepdims=True)
    lse_h = m2 + jnp.log(jnp.sum(jnp.exp(ch - m2), axis=0, keepdims=True))
    tgt_h = f[16:17, :].astype(jnp.int32)
    iota3 = jax.lax.broadcasted_iota(jnp.int32, (3, _BLKE), 0)
    chosen_h = jnp.sum(jnp.where(iota3 == tgt_h, ch, 0.0), axis=0, keepdims=True)
    charge_loss = lse_h - chosen_h

    dpos = f[17:20, :] - f[8:11, :]
    pos_loss = jnp.sqrt(jnp.sum(dpos * dpos, axis=0, keepdims=True))
    dmom = f[20:23, :] - f[11:14, :]
    mom_loss = jnp.sqrt(jnp.sum(dmom * dmom, axis=0, keepdims=True))

    de = f[14:15, :] - f[23:24, :]
    energy_loss = de * de

    o_ref[...] = class_loss + charge_loss + pos_loss + mom_loss + energy_loss


@functools.partial(jax.jit, static_argnums=())
def _pair_loss(feat):
    out = pl.pallas_call(
        _pair_loss_body,
        grid=(_G,),
        in_specs=[pl.BlockSpec((24, _BLKE), lambda i: (0, i))],
        out_specs=pl.BlockSpec((1, _BLKE), lambda i: (0, i)),
        out_shape=jax.ShapeDtypeStruct((1, _E), jnp.float32),
    )(feat)
    return out.reshape(_E)


def kernel(edge_index, particle_class, particle_charge, particle_pos,
           particle_mom, particle_energy, pflow_class_logits,
           pflow_charge_logits, pflow_pos, pflow_mom, pflow_energy,
           predicted_setsizes, particle_batch, pflow_batch):
    src = edge_index[0]
    dst = edge_index[1]

    ptable = jnp.concatenate([
        particle_class.astype(jnp.float32)[:, None],
        particle_charge.astype(jnp.float32)[:, None],
        particle_pos, particle_mom, particle_energy[:, None],
    ], axis=1)
    ftable = jnp.concatenate([
        pflow_class_logits, pflow_charge_logits,
        pflow_pos, pflow_mom, pflow_energy[:, None],
    ], axis=1)

    feat = jnp.concatenate([ftable[dst].T, ptable[src].T], axis=0)
    pair_loss = _pair_loss(feat)

    min_pflow = jax.ops.segment_min(pair_loss, dst, num_segments=_NF)
    min_pflow = jnp.where(jnp.isfinite(min_pflow), min_pflow, 0.0)
    min_part = jax.ops.segment_min(pair_loss, src, num_segments=_NP)
    min_part = jnp.where(jnp.isfinite(min_part), min_part, 0.0)

    particle_loss = jax.ops.segment_sum(min_part, particle_batch, num_segments=_B).mean()
    pflow_loss = jax.ops.segment_sum(min_pflow, pflow_batch, num_segments=_B).mean()

    n_per_event = jnp.bincount(particle_batch, length=_B)
    pf_counts = jnp.bincount(pflow_batch, length=_B)
    mean_setsizes = jax.ops.segment_sum(predicted_setsizes, pflow_batch, num_segments=_B)
    mean_setsizes = mean_setsizes / jnp.maximum(pf_counts, 1)[:, None].astype(jnp.float32)
    tgt = jnp.clip(n_per_event, 0, _S - 1)
    logp = jax.nn.log_softmax(mean_setsizes, axis=-1)
    setsize_loss = (-jnp.take_along_axis(logp, tgt[:, None], axis=-1)[:, 0]).mean()

    loss = particle_loss + pflow_loss + setsize_loss
    return loss, particle_loss, setsize_loss, pflow_loss


# speedup vs baseline: 6.4752x; 1.0002x over previous
import functools

import jax
import jax.numpy as jnp
from jax.experimental import pallas as pl

_NP = 100000
_NF = 100000
_E = 1600000
_B = 1000
_S = 256

_BLKE = 64000
_G = _E // _BLKE


def _pair_loss_body(f_ref, o_ref):
    f = f_ref[...]

    cl = f[0:5, :]
    m = jnp.max(cl, axis=0, keepdims=True)
    lse_c = m + jnp.log(jnp.sum(jnp.exp(cl - m), axis=0, keepdims=True))
    tgt_c = f[15:16, :].astype(jnp.int32)
    iota5 = jax.lax.broadcasted_iota(jnp.int32, (5, _BLKE), 0)
    chosen_c = jnp.sum(jnp.where(iota5 == tgt_c, cl, 0.0), axis=0, keepdims=True)
    class_loss = lse_c - chosen_c

    ch = f[5:8, :]
    m2 = jnp.max(ch, axis=0, keepdims=True)
    lse_h = m2 + jnp.log(jnp.sum(jnp.exp(ch - m2), axis=0, keepdims=True))
    tgt_h = f[16:17, :].astype(jnp.int32)
    iota3 = jax.lax.broadcasted_iota(jnp.int32, (3, _BLKE), 0)
    chosen_h = jnp.sum(jnp.where(iota3 == tgt_h, ch, 0.0), axis=0, keepdims=True)
    charge_loss = lse_h - chosen_h

    dpos = f[17:20, :] - f[8:11, :]
    pos_loss = jnp.sqrt(jnp.sum(dpos * dpos, axis=0, keepdims=True))
    dmom = f[20:23, :] - f[11:14, :]
    mom_loss = jnp.sqrt(jnp.sum(dmom * dmom, axis=0, keepdims=True))

    de = f[14:15, :] - f[23:24, :]
    energy_loss = de * de

    o_ref[...] = class_loss + charge_loss + pos_loss + mom_loss + energy_loss


@functools.partial(jax.jit, static_argnums=())
def _pair_loss(feat):
    out = pl.pallas_call(
        _pair_loss_body,
        grid=(_G,),
        in_specs=[pl.BlockSpec((24, _BLKE), lambda i: (0, i))],
        out_specs=pl.BlockSpec((1, _BLKE), lambda i: (0, i)),
        out_shape=jax.ShapeDtypeStruct((1, _E), jnp.float32),
    )(feat)
    return out.reshape(_E)


def kernel(edge_index, particle_class, particle_charge, particle_pos,
           particle_mom, particle_energy, pflow_class_logits,
           pflow_charge_logits, pflow_pos, pflow_mom, pflow_energy,
           predicted_setsizes, particle_batch, pflow_batch):
    src = edge_index[0]
    dst = edge_index[1]

    ptable = jnp.concatenate([
        particle_class.astype(jnp.float32)[:, None],
        particle_charge.astype(jnp.float32)[:, None],
        particle_pos, particle_mom, particle_energy[:, None],
    ], axis=1)
    ftable = jnp.concatenate([
        pflow_class_logits, pflow_charge_logits,
        pflow_pos, pflow_mom, pflow_energy[:, None],
    ], axis=1)

    feat = jnp.concatenate([ftable[dst].T, ptable[src].T], axis=0)
    pair_loss = _pair_loss(feat)

    min_pflow = jax.ops.segment_min(pair_loss, dst, num_segments=_NF)
    min_pflow = jnp.where(jnp.isfinite(min_pflow), min_pflow, 0.0)
    min_part = jax.ops.segment_min(pair_loss, src, num_segments=_NP)
    min_part = jnp.where(jnp.isfinite(min_part), min_part, 0.0)

    particle_loss = jax.ops.segment_sum(min_part, particle_batch, num_segments=_B).mean()
    pflow_loss = jax.ops.segment_sum(min_pflow, pflow_batch, num_segments=_B).mean()

    n_per_event = jnp.bincount(particle_batch, length=_B)
    pf_counts = jnp.bincount(pflow_batch, length=_B)
    mean_setsizes = jax.ops.segment_sum(predicted_setsizes, pflow_batch, num_segments=_B)
    mean_setsizes = mean_setsizes / jnp.maximum(pf_counts, 1)[:, None].astype(jnp.float32)
    tgt = jnp.clip(n_per_event, 0, _S - 1)
    logp = jax.nn.log_softmax(mean_setsizes, axis=-1)
    setsize_loss = (-jnp.take_along_axis(logp, tgt[:, None], axis=-1)[:, 0]).mean()

    loss = particle_loss + pflow_loss + setsize_loss
    return loss, particle_loss, setsize_loss, pflow_loss
